# Initial kernel scaffold; baseline (speedup 1.0000x reference)
import jax
import jax.numpy as jnp
from jax.experimental import pallas as pl


def kernel(inputs, expert_indices, W, bias):
    raise NotImplementedError("write your pallas kernel here")


# speedup vs baseline: 2.3822x; 2.3822x over previous
import functools

import jax
import jax.numpy as jnp
from jax import lax
from jax.experimental import pallas as pl
from jax.experimental.pallas import tpu as pltpu
from jax.experimental.pallas import tpu_sc as plsc

E = 16
K = 2
BLK = 256
NP = 8192
P = NP + E * BLK
NB = P // BLK

NC = 2
NS = 16
NW = NC * NS
PER_W = NP // NW
CHUNK = 64


def _cumsum_axis1(x):
    n = x.shape[1]
    s = 1
    while s < n:
        x = x + jnp.concatenate(
            [jnp.zeros((x.shape[0], s), x.dtype), x[:, :-s]], axis=1)
        s *= 2
    return x


def _routing_body(idx_ref, dest_ref, be_ref, tb_ref):
    idx = idx_ref[...]
    e_iota = lax.broadcasted_iota(jnp.int32, (E, NP), 0)
    oh = (jnp.broadcast_to(idx, (E, NP)) == e_iota).astype(jnp.int32)
    cum = _cumsum_axis1(oh)
    rank = jnp.sum(jnp.where(oh == 1, cum - 1, 0), axis=0, keepdims=True)
    counts = cum[:, NP - 1:NP]
    nbk = (counts + (BLK - 1)) // BLK
    cc = nbk
    s = 1
    while s < E:
        cc = cc + jnp.concatenate(
            [jnp.zeros((s, 1), jnp.int32), cc[:-s, :]], axis=0)
        s *= 2
    blk_off = cc - nbk
    tb_ref[...] = cc[E - 1:E, :]
    dest_ref[...] = jnp.sum(
        jnp.where(oh == 1, jnp.broadcast_to(blk_off * BLK, (E, NP)), 0),
        axis=0, keepdims=True) + rank
    b_iota = lax.broadcasted_iota(jnp.int32, (E, NB), 1)
    cmp = (b_iota >= jnp.broadcast_to(blk_off, (E, NB))).astype(jnp.int32)
    be = jnp.sum(cmp, axis=0, keepdims=True) - 1
    be_ref[...] = jnp.clip(be, 0, E - 1)


def _routing(idx2):
    return pl.pallas_call(
        _routing_body,
        out_shape=(
            jax.ShapeDtypeStruct((1, NP), jnp.int32),
            jax.ShapeDtypeStruct((1, NB), jnp.int32),
            jax.ShapeDtypeStruct((1, 1), jnp.int32),
        ),
    )(idx2)


def _dispatch_body(x_hbm, dest_hbm, xs_hbm, didx_v, sidx_v, rows_v, sem_g, sem_s):
    wid = lax.axis_index("s") * NC + lax.axis_index("c")
    base_w = wid * PER_W

    def chunk(i, carry):
        base = base_w + i * CHUNK
        pltpu.sync_copy(dest_hbm.at[pl.ds(base, CHUNK)], didx_v)
        for v in range(CHUNK // 16):
            vec = lax.shift_right_logical(
                base + v * 16 + lax.broadcasted_iota(jnp.int32, (16,), 0), 1)
            sidx_v[pl.ds(v * 16, 16)] = vec
        pltpu.async_copy(x_hbm.at[sidx_v], rows_v, sem_g).wait()
        pltpu.async_copy(rows_v, xs_hbm.at[didx_v], sem_s).wait()
        return carry

    lax.fori_loop(0, PER_W // CHUNK, chunk, 0)


def _dispatch(x, dest):
    D = x.shape[1]
    mesh = plsc.VectorSubcoreMesh(core_axis_name="c", subcore_axis_name="s")
    fn = pl.kernel(
        _dispatch_body,
        out_type=jax.ShapeDtypeStruct((P, D), jnp.float32),
        mesh=mesh,
        scratch_types=[
            pltpu.VMEM((CHUNK,), jnp.int32),
            pltpu.VMEM((CHUNK,), jnp.int32),
            pltpu.VMEM((CHUNK, D), jnp.float32),
            pltpu.SemaphoreType.DMA,
            pltpu.SemaphoreType.DMA,
        ],
    )
    return fn(x, dest)


def _matmul_body(be_ref, tb_ref, x_ref, w_ref, b_ref, o_ref):
    blk = pl.program_id(0)

    @pl.when(blk < tb_ref[0])
    def _():
        acc = lax.dot_general(
            x_ref[...], w_ref[0],
            dimension_numbers=(((1,), (1,)), ((), ())),
            preferred_element_type=jnp.float32)
        o_ref[...] = acc + b_ref[0]


def _matmul(xs, W, bias, be, tb):
    D = xs.shape[1]
    grid_spec = pltpu.PrefetchScalarGridSpec(
        num_scalar_prefetch=2,
        grid=(NB,),
        in_specs=[
            pl.BlockSpec((BLK, D),
                         lambda b, be, tb: (jnp.where(b < tb[0], b, 0), 0)),
            pl.BlockSpec((1, D, D), lambda b, be, tb: (be[b], 0, 0)),
            pl.BlockSpec((1, 1, D), lambda b, be, tb: (be[b], 0, 0)),
        ],
        out_specs=pl.BlockSpec((BLK, D), lambda b, be, tb: (b, 0)),
    )
    return pl.pallas_call(
        _matmul_body,
        grid_spec=grid_spec,
        out_shape=jax.ShapeDtypeStruct((P, D), jnp.float32),
    )(be, tb, xs, W, bias.reshape(E, 1, D))


def _combine_body(ys_hbm, dest_hbm, out_hbm, didx_v, rows_v, sem_g):
    wid = lax.axis_index("s") * NC + lax.axis_index("c")
    base_w = wid * PER_W

    def chunk(i, carry):
        base = base_w + i * CHUNK
        pltpu.sync_copy(dest_hbm.at[pl.ds(base, CHUNK)], didx_v)
        pltpu.async_copy(ys_hbm.at[didx_v], rows_v, sem_g).wait()
        pltpu.sync_copy(rows_v, out_hbm.at[pl.ds(base, CHUNK)])
        return carry

    lax.fori_loop(0, PER_W // CHUNK, chunk, 0)


def _combine(ys, dest):
    D = ys.shape[1]
    mesh = plsc.VectorSubcoreMesh(core_axis_name="c", subcore_axis_name="s")
    fn = pl.kernel(
        _combine_body,
        out_type=jax.ShapeDtypeStruct((NP, D), jnp.float32),
        mesh=mesh,
        scratch_types=[
            pltpu.VMEM((CHUNK,), jnp.int32),
            pltpu.VMEM((CHUNK, D), jnp.float32),
            pltpu.SemaphoreType.DMA,
        ],
    )
    return fn(ys, dest)


def kernel(inputs, expert_indices, W, bias):
    Bb, Ss, Dd = inputs.shape
    x = inputs.reshape(Bb * Ss, Dd)
    idx2 = expert_indices.reshape(1, NP).astype(jnp.int32)
    dest2, be2, tb2 = _routing(idx2)
    dest = dest2.reshape(NP)
    xs = _dispatch(x, dest)
    ys = _matmul(xs, W, bias, be2.reshape(NB), tb2.reshape(1))
    out = _combine(ys, dest)
    return out.reshape(Bb, Ss, K, Dd)
